# baseline (speedup 1.0000x reference)
import functools

import jax
import jax.numpy as jnp
from jax import lax
from jax.experimental import pallas as pl
from jax.experimental.pallas import tpu as pltpu
from jax.experimental.pallas import tpu_sc as plsc

EMB_DIM = 16
HIST = 50
NUM_CORES = 2
NUM_SUBCORES = 16
NUM_WORKERS = NUM_CORES * NUM_SUBCORES
PAIRS_PER_W = HIST * EMB_DIM // NUM_WORKERS


def _sc_gather(x4, t_flat):
    hist, br, bc = x4.shape
    batch = br * bc
    vocab = t_flat.shape[0] // EMB_DIM
    mesh = plsc.VectorSubcoreMesh(core_axis_name="c", subcore_axis_name="s")

    @functools.partial(
        pl.kernel,
        mesh=mesh,
        out_type=jax.ShapeDtypeStruct((hist, EMB_DIM, batch), jnp.float32),
        scratch_types=[
            pltpu.VMEM((br, bc), jnp.int32),
            pltpu.VMEM((batch,), jnp.int32),
            pltpu.VMEM((batch,), jnp.int32),
            pltpu.VMEM((batch,), jnp.float32),
            pltpu.VMEM((batch,), jnp.float32),
            pltpu.SemaphoreType.DMA,
            pltpu.SemaphoreType.DMA,
        ],
        compiler_params=pltpu.CompilerParams(needs_layout_passes=False),
    )
    def k(idx_hbm, tab_hbm, out_hbm, idx_v, fidx0, fidx1, dst0, dst1,
          sg0, sg1):
        wid = lax.axis_index("s") * NUM_CORES + lax.axis_index("c")
        d = lax.rem(wid, EMB_DIM)
        h0 = wid // EMB_DIM
        dbase = d * vocab
        fidx = (fidx0, fidx1)
        dst = (dst0, dst1)
        sg = (sg0, sg1)

        def prep(kp, p):
            h = h0 + 2 * kp
            pltpu.sync_copy(idx_hbm.at[h], idx_v)

            def row(r, carry):
                for s in range(bc // 16):
                    v = idx_v[r, pl.ds(s * 16, 16)]
                    fidx[p][pl.ds(r * bc + s * 16, 16)] = v + dbase
                return carry

            lax.fori_loop(0, br, row, 0)

        def start_gather(p):
            return pltpu.async_copy(tab_hbm.at[fidx[p]], dst[p], sg[p])

        def wait_gather(p):
            pltpu.make_async_copy(tab_hbm.at[fidx[p]], dst[p], sg[p]).wait()

        def write(kp, p):
            h = h0 + 2 * kp
            pltpu.sync_copy(dst[p], out_hbm.at[h, d])

        prep(0, 0)
        start_gather(0)

        def body(i, carry):
            for b in range(2):
                kp = 2 * i + 1 + b
                p = (1 + b) % 2
                prep(kp, p)
                wait_gather(1 - p)
                write(kp - 1, 1 - p)
                start_gather(p)
            return carry

        lax.fori_loop(0, (PAIRS_PER_W - 1) // 2, body, 0)

        wait_gather(0)
        write(PAIRS_PER_W - 1, 0)

    return k(x4, t_flat)


def kernel(x, table):
    batch, hist = x.shape
    x4 = x.T.reshape(hist, batch // 128, 128).astype(jnp.int32)
    t_flat = table.T.reshape(-1)
    out = _sc_gather(x4, t_flat)
    return jnp.transpose(out, (2, 0, 1))


# speedup vs baseline: 2.3330x; 2.3330x over previous
import functools

import jax
import jax.numpy as jnp
from jax import lax
from jax.experimental import pallas as pl
from jax.experimental.pallas import tpu as pltpu
from jax.experimental.pallas import tpu_sc as plsc

EMB_DIM = 16
HIST = 50
NUM_CORES = 2
NUM_SUBCORES = 16
NUM_WORKERS = NUM_CORES * NUM_SUBCORES


def _sc_gather(x4, table):
    hist, br, bc = x4.shape
    batch = br * bc
    slab = batch // NUM_WORKERS
    rpw = slab // bc
    mesh = plsc.VectorSubcoreMesh(core_axis_name="c", subcore_axis_name="s")

    @functools.partial(
        pl.kernel,
        mesh=mesh,
        out_type=jax.ShapeDtypeStruct((hist, EMB_DIM, batch), jnp.float32),
        scratch_types=[
            pltpu.VMEM((rpw, bc), jnp.int32),
            pltpu.VMEM((rpw, bc), jnp.int32),
            pltpu.VMEM((slab,), jnp.int32),
            pltpu.VMEM((slab,), jnp.int32),
            pltpu.VMEM((slab, EMB_DIM), jnp.float32),
            pltpu.VMEM((slab, EMB_DIM), jnp.float32),
            pltpu.VMEM((EMB_DIM, slab), jnp.float32),
            pltpu.VMEM((EMB_DIM, slab), jnp.float32),
            pltpu.SemaphoreType.DMA,
            pltpu.SemaphoreType.DMA,
            pltpu.SemaphoreType.DMA,
            pltpu.SemaphoreType.DMA,
        ],
        compiler_params=pltpu.CompilerParams(
            use_tc_tiling_on_sc=False, needs_layout_passes=False),
    )
    def k(idx_hbm, tab_hbm, out_hbm, isl0, isl1, idx0, idx1, rows0, rows1,
          ob0, ob1, sg0, sg1, sw0, sw1):
        wid = lax.axis_index("s") * NUM_CORES + lax.axis_index("c")
        r0 = wid * rpw
        b0 = wid * slab
        isl = (isl0, isl1)
        idx = (idx0, idx1)
        rows = (rows0, rows1)
        ob = (ob0, ob1)
        sg = (sg0, sg1)
        sw = (sw0, sw1)
        iota = lax.iota(jnp.int32, 16)
        cvec = [iota * 0 + c for c in range(EMB_DIM)]

        def prep(h, p):
            pltpu.sync_copy(idx_hbm.at[h, pl.ds(r0, rpw)], isl[p])
            for r in range(rpw):
                for s in range(bc // 16):
                    idx[p][pl.ds(r * bc + s * 16, 16)] = \
                        isl[p][r, pl.ds(s * 16, 16)]

        def start_gather(p):
            return pltpu.async_copy(tab_hbm.at[idx[p]], rows[p], sg[p])

        def wait_gather(p):
            pltpu.make_async_copy(tab_hbm.at[idx[p]], rows[p], sg[p]).wait()

        def transpose(p):
            for g in range(slab // 16):
                items16 = g * 16 + iota
                for c in range(EMB_DIM):
                    vals = plsc.load_gather(rows[p], [items16, cvec[c]])
                    ob[p][c, pl.ds(g * 16, 16)] = vals

        def start_wb(h, p):
            return pltpu.async_copy(
                ob[p], out_hbm.at[h, :, pl.ds(b0, slab)], sw[p])

        def wait_wb(p):
            pltpu.make_async_copy(
                ob[p], out_hbm.at[0, :, pl.ds(b0, slab)], sw[p]).wait()

        for h in range(2):
            prep(h, h)
            start_gather(h)

        def body(i, carry):
            for b in range(2):
                h = 2 * i + 2 + b

                wait_gather(b)

                @pl.when(i >= 1)
                def _():
                    wait_wb(b)

                transpose(b)
                start_wb(h - 2, b)
                prep(h, b)
                start_gather(b)
            return carry

        lax.fori_loop(0, (hist - 2) // 2, body, 0)

        for b in range(2):
            h = hist - 2 + b
            wait_gather(b)
            wait_wb(b)
            transpose(b)
            start_wb(h, b)
        for b in range(2):
            wait_wb(b)

    return k(x4, table)


def kernel(x, table):
    batch, hist = x.shape
    x4 = x.T.reshape(hist, batch // 128, 128).astype(jnp.int32)
    out = _sc_gather(x4, table)
    return jnp.transpose(out, (2, 0, 1))


# speedup vs baseline: 2.6602x; 1.1403x over previous
import functools

import jax
import jax.numpy as jnp
from jax import lax
from jax.experimental import pallas as pl
from jax.experimental.pallas import tpu as pltpu
from jax.experimental.pallas import tpu_sc as plsc

EMB_DIM = 16
HIST = 50
NUM_CORES = 2
NUM_SUBCORES = 16
NUM_WORKERS = NUM_CORES * NUM_SUBCORES


def _sc_gather(x4, table):
    hist, br, bc = x4.shape
    batch = br * bc
    slab = batch // NUM_WORKERS
    rpw = slab // bc
    mesh = plsc.VectorSubcoreMesh(core_axis_name="c", subcore_axis_name="s")

    @functools.partial(
        pl.kernel,
        mesh=mesh,
        out_type=jax.ShapeDtypeStruct((hist, EMB_DIM, batch), jnp.float32),
        scratch_types=[
            pltpu.VMEM((rpw, bc), jnp.int32),
            pltpu.VMEM((rpw, bc), jnp.int32),
            pltpu.VMEM((slab,), jnp.int32),
            pltpu.VMEM((slab,), jnp.int32),
            pltpu.VMEM((slab, EMB_DIM), jnp.float32),
            pltpu.VMEM((slab, EMB_DIM), jnp.float32),
            pltpu.VMEM((EMB_DIM, slab + 1), jnp.float32),
            pltpu.VMEM((EMB_DIM, slab + 1), jnp.float32),
            pltpu.SemaphoreType.DMA,
            pltpu.SemaphoreType.DMA,
            pltpu.SemaphoreType.DMA,
            pltpu.SemaphoreType.DMA,
        ],
        compiler_params=pltpu.CompilerParams(
            use_tc_tiling_on_sc=False, needs_layout_passes=False),
    )
    def k(idx_hbm, tab_hbm, out_hbm, isl0, isl1, idx0, idx1, rows0, rows1,
          ob0, ob1, sg0, sg1, sw0, sw1):
        wid = lax.axis_index("s") * NUM_CORES + lax.axis_index("c")
        r0 = wid * rpw
        b0 = wid * slab
        isl = (isl0, isl1)
        idx = (idx0, idx1)
        rows = (rows0, rows1)
        ob = (ob0, ob1)
        sg = (sg0, sg1)
        sw = (sw0, sw1)
        iota = lax.iota(jnp.int32, 16)

        def prep(h, p):
            pltpu.sync_copy(idx_hbm.at[h, pl.ds(r0, rpw)], isl[p])
            for r in range(rpw):
                for s in range(bc // 16):
                    idx[p][pl.ds(r * bc + s * 16, 16)] = \
                        isl[p][r, pl.ds(s * 16, 16)]

        def start_gather(p):
            return pltpu.async_copy(tab_hbm.at[idx[p]], rows[p], sg[p])

        def wait_gather(p):
            pltpu.make_async_copy(tab_hbm.at[idx[p]], rows[p], sg[p]).wait()

        def transpose(p):
            zero = iota * 0
            for i in range(slab):
                vals = rows[p][i, pl.ds(0, 16)]
                plsc.store_scatter(ob[p], [iota, zero + i], vals)

        def start_wb(h, p):
            return pltpu.async_copy(
                ob[p].at[:, pl.ds(0, slab)],
                out_hbm.at[h, :, pl.ds(b0, slab)], sw[p])

        def wait_wb(p):
            pltpu.make_async_copy(
                ob[p].at[:, pl.ds(0, slab)],
                out_hbm.at[0, :, pl.ds(b0, slab)], sw[p]).wait()

        for h in range(2):
            prep(h, h)
            start_gather(h)

        def body(i, carry):
            for b in range(2):
                h = 2 * i + 2 + b

                wait_gather(b)

                @pl.when(i >= 1)
                def _():
                    wait_wb(b)

                transpose(b)
                start_wb(h - 2, b)
                prep(h, b)
                start_gather(b)
            return carry

        lax.fori_loop(0, (hist - 2) // 2, body, 0)

        for b in range(2):
            h = hist - 2 + b
            wait_gather(b)
            wait_wb(b)
            transpose(b)
            start_wb(h, b)
        for b in range(2):
            wait_wb(b)

    return k(x4, table)


def kernel(x, table):
    batch, hist = x.shape
    x4 = x.T.reshape(hist, batch // 128, 128).astype(jnp.int32)
    out = _sc_gather(x4, table)
    return jnp.transpose(out, (2, 0, 1))


# speedup vs baseline: 2.6998x; 1.0149x over previous
import functools

import jax
import jax.numpy as jnp
from jax import lax
from jax.experimental import pallas as pl
from jax.experimental.pallas import tpu as pltpu
from jax.experimental.pallas import tpu_sc as plsc

EMB_DIM = 16
HIST = 50
NUM_CORES = 2
NUM_SUBCORES = 16
NUM_WORKERS = NUM_CORES * NUM_SUBCORES


def _sc_gather(x4, table):
    hist, br, bc = x4.shape
    batch = br * bc
    slab = batch // NUM_WORKERS
    rpw = slab // bc
    mesh = plsc.VectorSubcoreMesh(core_axis_name="c", subcore_axis_name="s")

    @functools.partial(
        pl.kernel,
        mesh=mesh,
        out_type=jax.ShapeDtypeStruct((hist, EMB_DIM, batch), jnp.float32),
        scratch_types=[
            pltpu.VMEM((rpw, bc), jnp.int32),
            pltpu.VMEM((rpw, bc), jnp.int32),
            pltpu.VMEM((slab,), jnp.int32),
            pltpu.VMEM((slab,), jnp.int32),
            pltpu.VMEM((slab, EMB_DIM), jnp.float32),
            pltpu.VMEM((slab, EMB_DIM), jnp.float32),
            pltpu.VMEM((EMB_DIM, slab + 1), jnp.float32),
            pltpu.VMEM((EMB_DIM, slab + 1), jnp.float32),
            pltpu.SemaphoreType.DMA,
            pltpu.SemaphoreType.DMA,
            pltpu.SemaphoreType.DMA,
            pltpu.SemaphoreType.DMA,
        ],
        compiler_params=pltpu.CompilerParams(
            use_tc_tiling_on_sc=False, needs_layout_passes=False),
    )
    def k(idx_hbm, tab_hbm, out_hbm, isl0, isl1, idx0, idx1, rows0, rows1,
          ob0, ob1, sg0, sg1, sw0, sw1):
        wid = lax.axis_index("s") * NUM_CORES + lax.axis_index("c")
        r0 = wid * rpw
        b0 = wid * slab
        isl = (isl0, isl1)
        idx = (idx0, idx1)
        rows = (rows0, rows1)
        ob = (ob0, ob1)
        sg = (sg0, sg1)
        sw = (sw0, sw1)
        iota = lax.iota(jnp.int32, 16)

        def prep(h, p):
            pltpu.sync_copy(idx_hbm.at[h, pl.ds(r0, rpw)], isl[p])
            for r in range(rpw):
                for s in range(bc // 16):
                    idx[p][pl.ds(r * bc + s * 16, 16)] = \
                        isl[p][r, pl.ds(s * 16, 16)]

        def start_gather(p):
            return pltpu.async_copy(tab_hbm.at[idx[p]], rows[p], sg[p])

        def wait_gather(p):
            pltpu.make_async_copy(tab_hbm.at[idx[p]], rows[p], sg[p]).wait()

        def transpose(p):
            ivec = iota * 0
            for g in range(slab // 8):
                vals = [rows[p][g * 8 + j, pl.ds(0, 16)] for j in range(8)]
                for j in range(8):
                    plsc.store_scatter(ob[p], [iota, ivec], vals[j])
                    ivec = ivec + 1

        def start_wb(h, p):
            return pltpu.async_copy(
                ob[p].at[:, pl.ds(0, slab)],
                out_hbm.at[h, :, pl.ds(b0, slab)], sw[p])

        def wait_wb(p):
            pltpu.make_async_copy(
                ob[p].at[:, pl.ds(0, slab)],
                out_hbm.at[0, :, pl.ds(b0, slab)], sw[p]).wait()

        for h in range(2):
            prep(h, h)
            start_gather(h)

        def body(i, carry):
            for b in range(2):
                h = 2 * i + 2 + b

                wait_gather(b)

                @pl.when(i >= 1)
                def _():
                    wait_wb(b)

                transpose(b)
                start_wb(h - 2, b)
                prep(h, b)
                start_gather(b)
            return carry

        lax.fori_loop(0, (hist - 2) // 2, body, 0)

        for b in range(2):
            h = hist - 2 + b
            wait_gather(b)
            wait_wb(b)
            transpose(b)
            start_wb(h, b)
        for b in range(2):
            wait_wb(b)

    return k(x4, table)


def kernel(x, table):
    batch, hist = x.shape
    x4 = x.T.reshape(hist, batch // 128, 128).astype(jnp.int32)
    out = _sc_gather(x4, table)
    return jnp.transpose(out, (2, 0, 1))
